# baseline (speedup 1.0000x reference)
import functools

import jax
import jax.numpy as jnp
from jax import lax
from jax.experimental import pallas as pl
from jax.experimental.pallas import tpu as pltpu
from jax.experimental.pallas import tpu_sc as plsc

_BATCH = 16384
_FIELDS = 26
_DIM = 32
_NC = 2
_NS = 16
_NW = _NC * _NS
_BPW = _BATCH // _NW
_NBUF = 4

_mesh = plsc.VectorSubcoreMesh(
    core_axis_name="c", subcore_axis_name="s", num_cores=_NC, num_subcores=_NS
)

_VOCAB = 1_000_000
_RB = 320
_NDB = _VOCAB // _RB
_DKMAX = -(-_NDB // _NW)


@functools.partial(
    pl.kernel,
    mesh=_mesh,
    out_type=jax.ShapeDtypeStruct((_VOCAB * _DIM,), jnp.float32),
    scratch_types=[
        pltpu.VMEM((_RB, _DIM), jnp.float32),
        pltpu.VMEM((_RB * _DIM,), jnp.float32),
        pltpu.SemaphoreType.DMA,
    ],
    compiler_params=pltpu.CompilerParams(use_tc_tiling_on_sc=True),
)
def _detile_kernel(trm_hbm, tlin_hbm, buf2, buf1, sem):
    wid = lax.axis_index("s") * _NC + lax.axis_index("c")

    @pl.loop(0, _DKMAX)
    def _blk(k):
        j = wid + _NW * k

        @pl.when(j < _NDB)
        def _():
            r0 = j * _RB
            pltpu.async_copy(trm_hbm.at[pl.ds(r0, _RB)], buf2, sem).wait()

            @pl.loop(0, _RB // 16)
            def _row16(t):
                for u in range(16):
                    r = t * 16 + u
                    buf1[pl.ds(r * _DIM, 16)] = buf2[r, pl.ds(0, 16)]
                    buf1[pl.ds(r * _DIM + 16, 16)] = buf2[r, pl.ds(16, 16)]

            pltpu.async_copy(buf1, tlin_hbm.at[pl.ds(r0 * _DIM, _RB * _DIM)], sem).wait()


@functools.partial(
    pl.kernel,
    mesh=_mesh,
    out_type=jax.ShapeDtypeStruct((_BATCH, _FIELDS, _DIM), jnp.float32),
    scratch_types=[
        pltpu.VMEM((_FIELDS, _BPW), jnp.int32),
        pltpu.VMEM((_NBUF, _BPW, _DIM), jnp.float32),
        pltpu.SemaphoreType.DMA,
    ],
    compiler_params=pltpu.CompilerParams(use_tc_tiling_on_sc=False),
)
def _gather_kernel(table_hbm, idxt_hbm, out_hbm, idx_v, rows_v, sem):
    wid = lax.axis_index("s") * _NC + lax.axis_index("c")
    b0 = wid * _BPW
    pltpu.sync_copy(idxt_hbm.at[:, pl.ds(b0, _BPW)], idx_v)

    for f in range(_NBUF - 1):
        pltpu.async_copy(table_hbm.at[idx_v.at[f]], rows_v.at[f], sem)

    @pl.loop(0, _FIELDS)
    def _field(f):
        b = lax.rem(f, _NBUF)
        pltpu.make_async_copy(table_hbm.at[idx_v.at[f]], rows_v.at[b], sem).wait()

        pltpu.sync_copy(rows_v.at[b], out_hbm.at[pl.ds(b0, _BPW), f])

        @pl.when(f + _NBUF - 1 < _FIELDS)
        def _():
            nxt = f + _NBUF - 1
            pltpu.async_copy(
                table_hbm.at[idx_v.at[nxt]], rows_v.at[lax.rem(nxt, _NBUF)], sem
            )


def kernel(x, table):
    table_lin = _detile_kernel(table).reshape(_VOCAB, _DIM)
    return _gather_kernel(table_lin, x.T.astype(jnp.int32))


# speedup vs baseline: 1.2623x; 1.2623x over previous
import functools

import jax
import jax.numpy as jnp
from jax import lax
from jax.experimental import pallas as pl
from jax.experimental.pallas import tpu as pltpu
from jax.experimental.pallas import tpu_sc as plsc

_BATCH = 16384
_FIELDS = 26
_DIM = 32
_NC = 2
_NS = 16
_NW = _NC * _NS
_BPW = _BATCH // _NW
_NBUF = 4

_mesh = plsc.VectorSubcoreMesh(
    core_axis_name="c", subcore_axis_name="s", num_cores=_NC, num_subcores=_NS
)

_VOCAB = 1_000_000
_RB = 320
_NDB = _VOCAB // _RB
_DKMAX = -(-_NDB // _NW)


@functools.partial(
    pl.kernel,
    mesh=_mesh,
    out_type=jax.ShapeDtypeStruct((_VOCAB * _DIM,), jnp.float32),
    scratch_types=[
        pltpu.VMEM((2, _RB, _DIM), jnp.float32),
        pltpu.VMEM((2, _RB * _DIM), jnp.float32),
        pltpu.SemaphoreType.DMA,
        pltpu.SemaphoreType.DMA,
    ],
    compiler_params=pltpu.CompilerParams(use_tc_tiling_on_sc=True),
)
def _detile_kernel(trm_hbm, tlin_hbm, buf2, buf1, sem_in, sem_out):
    wid = lax.axis_index("s") * _NC + lax.axis_index("c")
    nkd = jnp.where(wid < _NDB - _NW * (_DKMAX - 1), _DKMAX, _DKMAX - 1)

    def blk(k):
        return (wid + _NW * k) * _RB

    pltpu.async_copy(trm_hbm.at[pl.ds(blk(0), _RB)], buf2.at[0], sem_in)

    @pl.loop(0, _DKMAX)
    def _blk(k):
        @pl.when(k < nkd)
        def _():
            b = lax.rem(k, 2)
            pltpu.make_async_copy(
                trm_hbm.at[pl.ds(0, _RB)], buf2.at[b], sem_in
            ).wait()

            @pl.when(k + 1 < nkd)
            def _():
                pltpu.async_copy(
                    trm_hbm.at[pl.ds(blk(k + 1), _RB)], buf2.at[1 - b], sem_in
                )

            @pl.when(k >= 2)
            def _():
                pltpu.make_async_copy(
                    buf1.at[0], tlin_hbm.at[pl.ds(0, _RB * _DIM)], sem_out
                ).wait()

            for r in range(_RB):
                buf1[b, pl.ds(r * _DIM, 16)] = buf2[b, r, pl.ds(0, 16)]
                buf1[b, pl.ds(r * _DIM + 16, 16)] = buf2[b, r, pl.ds(16, 16)]

            pltpu.async_copy(
                buf1.at[b], tlin_hbm.at[pl.ds(blk(k) * _DIM, _RB * _DIM)], sem_out
            )

    @pl.when(nkd >= 2)
    def _():
        pltpu.make_async_copy(
            buf1.at[0], tlin_hbm.at[pl.ds(0, _RB * _DIM)], sem_out
        ).wait()
    pltpu.make_async_copy(
        buf1.at[0], tlin_hbm.at[pl.ds(0, _RB * _DIM)], sem_out
    ).wait()


@functools.partial(
    pl.kernel,
    mesh=_mesh,
    out_type=jax.ShapeDtypeStruct((_BATCH, _FIELDS, _DIM), jnp.float32),
    scratch_types=[
        pltpu.VMEM((_FIELDS, _BPW), jnp.int32),
        pltpu.VMEM((_NBUF, _BPW, _DIM), jnp.float32),
        pltpu.SemaphoreType.DMA,
    ],
    compiler_params=pltpu.CompilerParams(use_tc_tiling_on_sc=False),
)
def _gather_kernel(table_hbm, idxt_hbm, out_hbm, idx_v, rows_v, sem):
    wid = lax.axis_index("s") * _NC + lax.axis_index("c")
    b0 = wid * _BPW
    pltpu.sync_copy(idxt_hbm.at[:, pl.ds(b0, _BPW)], idx_v)

    for f in range(_NBUF - 1):
        pltpu.async_copy(table_hbm.at[idx_v.at[f]], rows_v.at[f], sem)

    @pl.loop(0, _FIELDS)
    def _field(f):
        b = lax.rem(f, _NBUF)
        pltpu.make_async_copy(table_hbm.at[idx_v.at[f]], rows_v.at[b], sem).wait()

        pltpu.sync_copy(rows_v.at[b], out_hbm.at[pl.ds(b0, _BPW), f])

        @pl.when(f + _NBUF - 1 < _FIELDS)
        def _():
            nxt = f + _NBUF - 1
            pltpu.async_copy(
                table_hbm.at[idx_v.at[nxt]], rows_v.at[lax.rem(nxt, _NBUF)], sem
            )


def kernel(x, table):
    table_lin = _detile_kernel(table).reshape(_VOCAB, _DIM)
    return _gather_kernel(table_lin, x.T.astype(jnp.int32))
